# baseline (speedup 1.0000x reference)
import jax
import jax.numpy as jnp
from jax.experimental import pallas as pl

_N, _T, _P, _D = 8, 48, 325, 512
_M = _N * _T * _P

_BM = 1920
_C = _BM // 128
_GRID = _M // _BM


def _body(x_ref, m_ref, b_ref, out_ref):
    xt = jnp.transpose(x_ref[...])
    bad = jnp.isnan(xt[0:1, :])
    t = xt[1:2, :]
    mon = jnp.concatenate([m_ref[c:c + 1, :] for c in range(_C)],
                          axis=1) > 0.5
    p_valid = mon & ~bad
    coef = jnp.where(p_valid, xt[0:1, :], 0.0)
    one = jnp.ones((1, _BM), jnp.float32)
    zero = jnp.zeros((1, _BM), jnp.float32)
    a = jnp.concatenate(
        [t, coef, p_valid.astype(jnp.float32), (mon & bad).astype(jnp.float32),
         (~mon).astype(jnp.float32), one, zero, zero], axis=0)
    out_ref[...] = jax.lax.dot_general(
        a.astype(jnp.bfloat16), b_ref[...], (((0,), (0,)), ((), ())),
        preferred_element_type=jnp.float32)


def kernel(x, monitor_mask, time_emb_w, time_emb_b, value_emb_w, value_emb_b,
           empty_token, unmonitored_token):
    f32 = jnp.float32
    xm = x.reshape(_M, 2)
    mm = jnp.pad(monitor_mask.reshape(_GRID, _C, 128).astype(f32),
                 ((0, 0), (0, 16 - _C), (0, 0))).reshape(_GRID * 16, 128)

    b = jnp.concatenate([time_emb_w.reshape(1, _D),
                         value_emb_w.reshape(1, _D),
                         value_emb_b.reshape(1, _D),
                         empty_token.reshape(1, _D),
                         unmonitored_token.reshape(1, _D),
                         time_emb_b.reshape(1, _D),
                         jnp.zeros((2, _D), f32)], axis=0
                        ).astype(jnp.bfloat16)

    out = pl.pallas_call(
        _body,
        grid=(_GRID,),
        in_specs=[pl.BlockSpec((_BM, 2), lambda i: (i, 0)),
                  pl.BlockSpec((16, 128), lambda i: (i, 0)),
                  pl.BlockSpec((8, _D), lambda i: (0, 0))],
        out_specs=pl.BlockSpec((_BM, _D), lambda i: (i, 0)),
        out_shape=jax.ShapeDtypeStruct((_M, _D), jnp.float32),
    )(xm, mm, b)
    return out.reshape(_N, _T, _P, _D)


# speedup vs baseline: 1.8056x; 1.8056x over previous
import jax
import jax.numpy as jnp
from jax.experimental import pallas as pl

_N, _T, _P, _D = 8, 48, 325, 512
_S = _N * _T
_R = 8
_GRID = _S // _R


def _body(x_ref, m_ref, tw_ref, c_ref, vw_ref, et_ref, ut_ref, out_ref):
    tw = tw_ref[...]
    c = c_ref[...]
    vw = vw_ref[...]
    et = et_ref[...]
    ut = ut_ref[...]
    mt = jnp.transpose(m_ref[...])
    for r in range(_R):
        xv = x_ref[r]
        v = xv[:, 0:1]
        t = xv[:, 1:2]
        mon = mt[:, r:r + 1] > 0.5
        bad = jnp.isnan(v)
        ve = jnp.where(bad, et, v * vw)
        ve = jnp.where(mon, ve, ut)
        out_ref[r] = t * tw + c + ve


def kernel(x, monitor_mask, time_emb_w, time_emb_b, value_emb_w, value_emb_b,
           empty_token, unmonitored_token):
    xs = x.reshape(_S, _P, 2)
    ms = monitor_mask.reshape(_S, _P).astype(jnp.float32)
    c = time_emb_b + value_emb_b
    et = empty_token.reshape(1, _D) - value_emb_b
    ut = unmonitored_token.reshape(1, _D) - value_emb_b

    out = pl.pallas_call(
        _body,
        grid=(_GRID,),
        in_specs=[pl.BlockSpec((_R, _P, 2), lambda i: (i, 0, 0)),
                  pl.BlockSpec((_R, _P), lambda i: (i, 0)),
                  pl.BlockSpec((1, _D), lambda i: (0, 0)),
                  pl.BlockSpec((1, _D), lambda i: (0, 0)),
                  pl.BlockSpec((1, _D), lambda i: (0, 0)),
                  pl.BlockSpec((1, _D), lambda i: (0, 0)),
                  pl.BlockSpec((1, _D), lambda i: (0, 0))],
        out_specs=pl.BlockSpec((_R, _P, _D), lambda i: (i, 0, 0)),
        out_shape=jax.ShapeDtypeStruct((_S, _P, _D), jnp.float32),
    )(xs, ms, time_emb_w, c, value_emb_w, et, ut)
    return out.reshape(_N, _T, _P, _D)


# speedup vs baseline: 2.0052x; 1.1105x over previous
import jax
import jax.numpy as jnp
from jax.experimental import pallas as pl

_N, _T, _P, _D = 8, 48, 325, 512
_R = 8
_J = _T // _R


def _body(x_ref, m_ref, tw_ref, c_ref, vw_ref, et_ref, ut_ref, out_ref):
    tw = tw_ref[...]
    c = c_ref[...]
    vw = vw_ref[...]
    et = et_ref[...]
    ut = ut_ref[...]
    mt = jnp.transpose(m_ref[0])
    for r in range(_R):
        xv = x_ref[0, r]
        v = xv[:, 0:1]
        t = xv[:, 1:2]
        mon = mt[:, r:r + 1] > 0.5
        bad = jnp.isnan(v)
        ve = jnp.where(bad, et, v * vw)
        ve = jnp.where(mon, ve, ut)
        out_ref[0, r] = t * tw + c + ve


def kernel(x, monitor_mask, time_emb_w, time_emb_b, value_emb_w, value_emb_b,
           empty_token, unmonitored_token):
    ms = monitor_mask.astype(jnp.float32)
    c = time_emb_b + value_emb_b
    et = empty_token.reshape(1, _D) - value_emb_b
    ut = unmonitored_token.reshape(1, _D) - value_emb_b

    return pl.pallas_call(
        _body,
        grid=(_N, _J),
        in_specs=[pl.BlockSpec((1, _R, _P, 2), lambda n, j: (n, j, 0, 0)),
                  pl.BlockSpec((1, _R, _P), lambda n, j: (n, j, 0)),
                  pl.BlockSpec((1, _D), lambda n, j: (0, 0)),
                  pl.BlockSpec((1, _D), lambda n, j: (0, 0)),
                  pl.BlockSpec((1, _D), lambda n, j: (0, 0)),
                  pl.BlockSpec((1, _D), lambda n, j: (0, 0)),
                  pl.BlockSpec((1, _D), lambda n, j: (0, 0))],
        out_specs=pl.BlockSpec((1, _R, _P, _D), lambda n, j: (n, j, 0, 0)),
        out_shape=jax.ShapeDtypeStruct((_N, _T, _P, _D), jnp.float32),
    )(x, ms, time_emb_w, c, value_emb_w, et, ut)
